# Initial kernel scaffold; baseline (speedup 1.0000x reference)
import jax
import jax.numpy as jnp
from jax.experimental import pallas as pl


def kernel(batch_box_preds, batch_cls_preds):
    raise NotImplementedError("write your pallas kernel here")


# speedup vs baseline: 8.8794x; 8.8794x over previous
import functools

import jax
import jax.numpy as jnp
from jax import lax
from jax.experimental import pallas as pl

B, N, C = 4, 20000, 3
PRE, POST, TH = 1024, 512, 0.7


def _nms_body(bx_cn_ref, payload_ref, out_ref):
    x = bx_cn_ref[0:1, :]
    y = bx_cn_ref[1:2, :]
    dx = bx_cn_ref[2:3, :]
    dy = bx_cn_ref[3:4, :]
    ry = bx_cn_ref[4:5, :]
    c = jnp.abs(jnp.cos(ry))
    s = jnp.abs(jnp.sin(ry))
    hx = (dx * c + dy * s) * 0.5
    hy = (dx * s + dy * c) * 0.5
    a_row = jnp.concatenate(
        [x - hx, y - hy, x + hx, y + hy, jnp.zeros((4, PRE), jnp.float32)], axis=0
    )
    a_col = jnp.transpose(a_row)

    x1 = jnp.maximum(a_col[:, 0:1], a_row[0:1, :])
    y1 = jnp.maximum(a_col[:, 1:2], a_row[1:2, :])
    x2 = jnp.minimum(a_col[:, 2:3], a_row[2:3, :])
    y2 = jnp.minimum(a_col[:, 3:4], a_row[3:4, :])
    inter = jnp.clip(x2 - x1, 0.0, None) * jnp.clip(y2 - y1, 0.0, None)
    area_row = (a_row[2:3, :] - a_row[0:1, :]) * (a_row[3:4, :] - a_row[1:2, :])
    area_col = (a_col[:, 2:3] - a_col[:, 0:1]) * (a_col[:, 3:4] - a_col[:, 1:2])
    union = area_col + area_row - inter
    iou = inter / (union + 1e-6)

    row_i = lax.broadcasted_iota(jnp.int32, (PRE, PRE), 0)
    col_i = lax.broadcasted_iota(jnp.int32, (PRE, PRE), 1)
    tri = (row_i < col_i).astype(jnp.float32)
    sup = jnp.where(iou > TH, tri, 0.0)

    def cond(carry):
        _, changed = carry
        return changed

    def body(carry):
        keep, _ = carry
        v = jax.lax.dot_general(
            keep, sup, (((1,), (0,)), ((), ())),
            preferred_element_type=jnp.float32,
        )
        new = (v < 0.5).astype(jnp.float32)
        changed = jnp.sum(jnp.abs(new - keep)) > 0.0
        return new, changed

    keep0 = jnp.ones((1, PRE), jnp.float32)
    keep, _ = lax.while_loop(cond, body, (keep0, jnp.bool_(True)))

    pos = jax.lax.dot_general(
        keep, tri, (((1,), (0,)), ((), ())), preferred_element_type=jnp.float32,
    )
    slot = lax.broadcasted_iota(jnp.int32, (POST, PRE), 0).astype(jnp.float32)
    sel = jnp.where(
        (jnp.abs(pos - slot) < 0.5) & (keep > 0.5), 1.0, 0.0
    )
    out_ref[...] = jax.lax.dot_general(
        sel, payload_ref[...], (((1,), (0,)), ((), ())),
        preferred_element_type=jnp.float32,
    )


@jax.jit
def kernel(batch_box_preds, batch_cls_preds):
    scores = jnp.max(batch_cls_preds, axis=-1)
    labels = jnp.argmax(batch_cls_preds, axis=-1)
    sc, idx = lax.top_k(scores, PRE)
    bx = jnp.take_along_axis(batch_box_preds, idx[..., None], axis=1)
    lb = jnp.take_along_axis(labels, idx, axis=1)

    payload = jnp.concatenate(
        [bx, sc[..., None], lb.astype(jnp.float32)[..., None],
         jnp.zeros((B, PRE, 7), jnp.float32)], axis=-1,
    )
    bx_cn = jnp.concatenate(
        [bx[..., 0:1], bx[..., 1:2], bx[..., 3:4], bx[..., 4:5], bx[..., 6:7],
         jnp.zeros((B, PRE, 3), jnp.float32)], axis=-1,
    )
    bx_cn = jnp.swapaxes(bx_cn, 1, 2)

    grid = (B,)
    out = pl.pallas_call(
        _nms_body,
        grid=grid,
        in_specs=[
            pl.BlockSpec((None, 8, PRE), lambda b: (b, 0, 0)),
            pl.BlockSpec((None, PRE, 16), lambda b: (b, 0, 0)),
        ],
        out_specs=pl.BlockSpec((None, POST, 16), lambda b: (b, 0, 0)),
        out_shape=jax.ShapeDtypeStruct((B, POST, 16), jnp.float32),
    )(bx_cn, payload)

    rois = out[..., :7]
    roi_scores = out[..., 7]
    roi_labels = jnp.round(out[..., 8]).astype(jnp.int32) + 1
    return rois, roi_scores, roi_labels
